# Initial kernel scaffold; baseline (speedup 1.0000x reference)
import jax
import jax.numpy as jnp
from jax.experimental import pallas as pl


def kernel(user_id, prev_id, pos_id, neg_id, poi_weight, user_weight, user_global_weight, poi_bias_weight):
    raise NotImplementedError("write your pallas kernel here")


# speedup vs baseline: 7.2513x; 7.2513x over previous
import functools

import jax
import jax.numpy as jnp
from jax import lax
from jax.experimental import pallas as pl
from jax.experimental.pallas import tpu as pltpu
from jax.experimental.pallas import tpu_sc as plsc

N_POI = 1000
N_POI_PAD = 1024
N_USERS = 100
DIM = 64
BATCH = 16384
NUM_TILES = 32
B_PER_TILE = BATCH // NUM_TILES
CHUNKS = B_PER_TILE // 16


def _sc_body(poi_h, vtab_h, bias_h, uid_h, pid_h, qid_h, nid_h,
             d2p_h, d2n_h, bd_h, mask_h,
             poi_v, vtab_v, bias_v, uid_v, pid_v, qid_v, nid_v,
             outp_v, outn_v, outb_v, mask_v):
  c = lax.axis_index("c")
  s = lax.axis_index("s")
  wid = s * 2 + c
  base = wid * B_PER_TILE

  pltpu.sync_copy(poi_h, poi_v)
  pltpu.sync_copy(vtab_h, vtab_v)
  pltpu.sync_copy(bias_h, bias_v)
  pltpu.sync_copy(uid_h.at[pl.ds(base, B_PER_TILE)], uid_v)
  pltpu.sync_copy(pid_h.at[pl.ds(base, B_PER_TILE)], pid_v)
  pltpu.sync_copy(qid_h.at[pl.ds(base, B_PER_TILE)], qid_v)
  pltpu.sync_copy(nid_h.at[pl.ds(base, B_PER_TILE)], nid_v)

  zeros16 = jnp.zeros((16,), jnp.float32)
  for i in range(N_POI_PAD // 16):
    mask_v[pl.ds(i * 16, 16)] = zeros16

  ones16 = jnp.ones((16,), jnp.float32)

  def chunk(i, carry):
    sl = pl.ds(i * 16, 16)
    u = uid_v[sl]
    p = pid_v[sl]
    q = qid_v[sl]
    r = nid_v[sl]
    bq = plsc.load_gather(bias_v, [q])
    br = plsc.load_gather(bias_v, [r])
    ub = u * DIM
    pb = p * DIM
    qb = q * DIM
    rb = r * DIM
    accp = jnp.zeros((16,), jnp.float32)
    accn = jnp.zeros((16,), jnp.float32)
    for d in range(DIM):
      td = plsc.load_gather(poi_v, [pb + d]) + plsc.load_gather(vtab_v, [ub + d])
      ep = td - plsc.load_gather(poi_v, [qb + d])
      en = td - plsc.load_gather(poi_v, [rb + d])
      accp = accp + ep * ep
      accn = accn + en * en
    outp_v[sl] = accp
    outn_v[sl] = accn
    outb_v[sl] = bq - br
    plsc.store_scatter(mask_v, [p], ones16)
    plsc.store_scatter(mask_v, [q], ones16)
    plsc.store_scatter(mask_v, [r], ones16)
    return carry

  lax.fori_loop(0, CHUNKS, chunk, 0)

  pltpu.sync_copy(outp_v, d2p_h.at[pl.ds(base, B_PER_TILE)])
  pltpu.sync_copy(outn_v, d2n_h.at[pl.ds(base, B_PER_TILE)])
  pltpu.sync_copy(outb_v, bd_h.at[pl.ds(base, B_PER_TILE)])
  pltpu.sync_copy(mask_v, mask_h.at[wid])


_sc_kernel = functools.partial(
    pl.kernel,
    out_type=(
        jax.ShapeDtypeStruct((BATCH,), jnp.float32),
        jax.ShapeDtypeStruct((BATCH,), jnp.float32),
        jax.ShapeDtypeStruct((BATCH,), jnp.float32),
        jax.ShapeDtypeStruct((NUM_TILES, N_POI_PAD), jnp.float32),
    ),
    mesh=plsc.VectorSubcoreMesh(core_axis_name="c", subcore_axis_name="s"),
    compiler_params=pltpu.CompilerParams(needs_layout_passes=False),
    scratch_types=[
        pltpu.VMEM((N_POI_PAD * DIM,), jnp.float32),
        pltpu.VMEM((N_USERS * DIM,), jnp.float32),
        pltpu.VMEM((N_POI_PAD,), jnp.float32),
        pltpu.VMEM((B_PER_TILE,), jnp.int32),
        pltpu.VMEM((B_PER_TILE,), jnp.int32),
        pltpu.VMEM((B_PER_TILE,), jnp.int32),
        pltpu.VMEM((B_PER_TILE,), jnp.int32),
        pltpu.VMEM((B_PER_TILE,), jnp.float32),
        pltpu.VMEM((B_PER_TILE,), jnp.float32),
        pltpu.VMEM((B_PER_TILE,), jnp.float32),
        pltpu.VMEM((N_POI_PAD,), jnp.float32),
    ],
)(_sc_body)


def _tc_body(poi_ref, masks_ref, d2p_ref, d2n_ref, bd_ref, w_ref, obj_ref):
  m = jnp.max(masks_ref[...], axis=0)
  poi = poi_ref[...]
  n2 = jnp.sum(poi * poi, axis=1)
  denom = jnp.maximum(1.0, jnp.sqrt(n2))
  scale = jnp.where(m > 0.0, 1.0 / denom, 1.0)
  w_ref[...] = poi * scale[:, None]
  obj_ref[...] = bd_ref[...] - jnp.sqrt(d2p_ref[...]) + jnp.sqrt(d2n_ref[...])


def kernel(user_id, prev_id, pos_id, neg_id, poi_weight, user_weight,
           user_global_weight, poi_bias_weight):
  uid = user_id.astype(jnp.int32)
  pid = prev_id.astype(jnp.int32)
  qid = pos_id.astype(jnp.int32)
  nid = neg_id.astype(jnp.int32)
  poi_p = jnp.zeros((N_POI_PAD, DIM), jnp.float32).at[:N_POI].set(poi_weight)
  vtab = user_weight + user_global_weight
  bias_p = jnp.zeros((N_POI_PAD,), jnp.float32).at[:N_POI].set(
      poi_bias_weight[:, 0])

  d2p, d2n, bd, masks = _sc_kernel(poi_p.reshape(-1), vtab.reshape(-1),
                                   bias_p, uid, pid, qid, nid)

  w_p, obj2d = pl.pallas_call(
      _tc_body,
      out_shape=(
          jax.ShapeDtypeStruct((N_POI_PAD, DIM), jnp.float32),
          jax.ShapeDtypeStruct((128, 128), jnp.float32),
      ),
  )(poi_p, masks, d2p.reshape(128, 128), d2n.reshape(128, 128),
    bd.reshape(128, 128))

  return obj2d.reshape(BATCH), w_p[:N_POI]
